# baseline (speedup 1.0000x reference)
import functools

import jax
import jax.numpy as jnp
from jax import lax
from jax.experimental import pallas as pl
from jax.experimental.pallas import tpu as pltpu
from jax.experimental.pallas import tpu_sc as plsc

_START_IDX = 16384
_LANES = 16
_NUM_CORES = 2
_NUM_SUBCORES = 16
_NUM_WORKERS = _NUM_CORES * _NUM_SUBCORES
_BLOCK = 32768


@functools.lru_cache(maxsize=None)
def _build_sc_offset_add(total: int, every_other_lane: bool):
    nb_full = total // _BLOCK
    tail = total - nb_full * _BLOCK
    max_blocks = -(-nb_full // _NUM_WORKERS)
    tail_vecs = -(-tail // _LANES)
    mesh = plsc.VectorSubcoreMesh(core_axis_name="c", subcore_axis_name="s")

    @functools.partial(
        pl.kernel,
        out_type=jax.ShapeDtypeStruct((total,), jnp.int32),
        mesh=mesh,
        scratch_types=[pltpu.VMEM((_BLOCK,), jnp.int32)],
    )
    def run(x_hbm, o_hbm, buf):
        wid = lax.axis_index("s") * _NUM_CORES + lax.axis_index("c")
        if every_other_lane:
            lane = lax.iota(jnp.int32, _LANES)
            pat = jnp.where(lane % 2 == 0, _START_IDX, 0)
        else:
            pat = jnp.full((_LANES,), _START_IDX, dtype=jnp.int32)

        def add_vectors(nvec):
            def body(i, carry):
                sl = pl.ds(i * _LANES, _LANES)
                buf[sl] = buf[sl] + pat
                return carry

            lax.fori_loop(0, nvec, body, 0)

        for k in range(max_blocks):
            blk = wid + k * _NUM_WORKERS

            @pl.when(blk < nb_full)
            def _process():
                base = blk * _BLOCK
                pltpu.sync_copy(x_hbm.at[pl.ds(base, _BLOCK)], buf)
                add_vectors(_BLOCK // _LANES)
                pltpu.sync_copy(buf, o_hbm.at[pl.ds(base, _BLOCK)])

        if tail:

            @pl.when(wid == _NUM_WORKERS - 1)
            def _process_tail():
                base = nb_full * _BLOCK
                pltpu.sync_copy(x_hbm.at[pl.ds(base, tail)], buf.at[pl.ds(0, tail)])
                add_vectors(tail_vecs)
                pltpu.sync_copy(buf.at[pl.ds(0, tail)], o_hbm.at[pl.ds(base, tail)])

    return run


def kernel(indices, values):
    nnz = indices.shape[0]
    if indices.dtype == jnp.int32:
        total = nnz * 2
        flat = indices.reshape(total)
        out = _build_sc_offset_add(total, False)(flat)
        return out.reshape(nnz, 2), values
    words = lax.bitcast_convert_type(indices, jnp.int32)
    total = nnz * 4
    flat = words.reshape(total)
    out = _build_sc_offset_add(total, True)(flat)
    new_indices = lax.bitcast_convert_type(out.reshape(nnz, 2, 2), indices.dtype)
    return new_indices, values


# speedup vs baseline: 1.0114x; 1.0114x over previous
import functools

import jax
import jax.numpy as jnp
from jax import lax
from jax.experimental import pallas as pl
from jax.experimental.pallas import tpu as pltpu
from jax.experimental.pallas import tpu_sc as plsc

_START_IDX = 16384
_LANES = 16
_NUM_CORES = 2
_NUM_SUBCORES = 16
_NUM_WORKERS = _NUM_CORES * _NUM_SUBCORES
_BLOCK = 65520
_BUF = 65528


@functools.lru_cache(maxsize=None)
def _build_sc_offset_add(total: int, every_other_lane: bool):
    nb_full = total // _BLOCK
    tail = total - nb_full * _BLOCK
    max_blocks = -(-nb_full // _NUM_WORKERS)
    tail_vecs = -(-tail // _LANES)
    mesh = plsc.VectorSubcoreMesh(core_axis_name="c", subcore_axis_name="s")

    @functools.partial(
        pl.kernel,
        out_type=jax.ShapeDtypeStruct((total,), jnp.int32),
        mesh=mesh,
        scratch_types=[
            pltpu.VMEM((_BUF,), jnp.int32),
            pltpu.VMEM((_BUF,), jnp.int32),
            pltpu.SemaphoreType.DMA,
            pltpu.SemaphoreType.DMA,
            pltpu.SemaphoreType.DMA,
            pltpu.SemaphoreType.DMA,
        ],
    )
    def run(x_hbm, o_hbm, buf0, buf1, in0, in1, out0, out1):
        wid = lax.axis_index("s") * _NUM_CORES + lax.axis_index("c")
        bufs = [buf0, buf1]
        in_sems = [in0, in1]
        out_sems = [out0, out1]
        if every_other_lane:
            lane = lax.iota(jnp.int32, _LANES)
            pat = jnp.where(lane % 2 == 0, _START_IDX, 0)
        else:
            pat = jnp.full((_LANES,), _START_IDX, dtype=jnp.int32)

        def add_vectors(buf, nvec):
            @plsc.parallel_loop(0, nvec * _LANES, _LANES, unroll=8)
            def _(i):
                sl = pl.ds(i, _LANES)
                buf[sl] = buf[sl] + pat

        def blk_of(k):
            return wid + k * _NUM_WORKERS

        def in_copy(k, b):
            base = blk_of(k) * _BLOCK
            return pltpu.make_async_copy(
                x_hbm.at[pl.ds(base, _BLOCK)], bufs[b].at[pl.ds(0, _BLOCK)], in_sems[b]
            )

        def out_copy(k, b):
            base = blk_of(k) * _BLOCK
            return pltpu.make_async_copy(
                bufs[b].at[pl.ds(0, _BLOCK)], o_hbm.at[pl.ds(base, _BLOCK)], out_sems[b]
            )

        for k in range(max_blocks + 1):
            if k < max_blocks:
                b = k % 2

                @pl.when(blk_of(k) < nb_full)
                def _start_in():
                    if k >= 2:
                        out_copy(k - 2, b).wait()
                    in_copy(k, b).start()

            if k >= 1:
                p = (k - 1) % 2

                @pl.when(blk_of(k - 1) < nb_full)
                def _process():
                    in_copy(k - 1, p).wait()
                    add_vectors(bufs[p], _BLOCK // _LANES)
                    out_copy(k - 1, p).start()

        for k in (max_blocks - 2, max_blocks - 1):
            if k >= 0:

                @pl.when(blk_of(k) < nb_full)
                def _drain():
                    out_copy(k, k % 2).wait()

        if tail:

            @pl.when(wid == _NUM_WORKERS - 1)
            def _process_tail():
                base = nb_full * _BLOCK
                pltpu.sync_copy(x_hbm.at[pl.ds(base, tail)], buf0.at[pl.ds(0, tail)])
                add_vectors(buf0, tail_vecs)
                pltpu.sync_copy(buf0.at[pl.ds(0, tail)], o_hbm.at[pl.ds(base, tail)])

    return run


def kernel(indices, values):
    nnz = indices.shape[0]
    if indices.dtype == jnp.int32:
        total = nnz * 2
        flat = indices.reshape(total)
        out = _build_sc_offset_add(total, False)(flat)
        return out.reshape(nnz, 2), values
    words = lax.bitcast_convert_type(indices, jnp.int32)
    total = nnz * 4
    flat = words.reshape(total)
    out = _build_sc_offset_add(total, True)(flat)
    new_indices = lax.bitcast_convert_type(out.reshape(nnz, 2, 2), indices.dtype)
    return new_indices, values


# speedup vs baseline: 5.6512x; 5.5876x over previous
import functools

import jax
import jax.numpy as jnp
from jax import lax
from jax.experimental import pallas as pl
from jax.experimental.pallas import tpu as pltpu
from jax.experimental.pallas import tpu_sc as plsc

_START_IDX = 16384
_LANES = 16
_NUM_CORES = 2
_NUM_SUBCORES = 16
_NUM_WORKERS = _NUM_CORES * _NUM_SUBCORES
_BLOCK = 65520
_BUF = 65528


@functools.lru_cache(maxsize=None)
def _build_sc_offset_add(total: int, every_other_lane: bool):
    nb_full = total // _BLOCK
    tail = total - nb_full * _BLOCK
    max_blocks = -(-nb_full // _NUM_WORKERS)
    tail_vecs = -(-tail // _LANES)
    mesh = plsc.VectorSubcoreMesh(core_axis_name="c", subcore_axis_name="s")

    @functools.partial(
        pl.kernel,
        out_type=jax.ShapeDtypeStruct((total,), jnp.int32),
        mesh=mesh,
        scratch_types=[
            pltpu.VMEM((_BUF,), jnp.int32),
            pltpu.VMEM((_BUF,), jnp.int32),
            pltpu.SemaphoreType.DMA,
            pltpu.SemaphoreType.DMA,
            pltpu.SemaphoreType.DMA,
            pltpu.SemaphoreType.DMA,
        ],
    )
    def run(x_hbm, o_hbm, buf0, buf1, in0, in1, out0, out1):
        wid = lax.axis_index("s") * _NUM_CORES + lax.axis_index("c")
        bufs = [buf0, buf1]
        in_sems = [in0, in1]
        out_sems = [out0, out1]
        if every_other_lane:
            lane = lax.iota(jnp.int32, _LANES)
            pat = jnp.where(lane % 2 == 0, _START_IDX, 0)
        else:
            pat = jnp.full((_LANES,), _START_IDX, dtype=jnp.int32)

        def add_vectors(buf, nvec):
            @plsc.parallel_loop(0, nvec * _LANES, _LANES, unroll=8)
            def _(i):
                sl = pl.ds(i, _LANES)
                buf[sl] = buf[sl] + pat

        def blk_of(k):
            return wid + k * _NUM_WORKERS

        def in_copy(k, b):
            base = blk_of(k) * _BLOCK
            return pltpu.make_async_copy(
                x_hbm.at[pl.ds(base, _BLOCK)], bufs[b].at[pl.ds(0, _BLOCK)], in_sems[b]
            )

        def out_copy(k, b):
            base = blk_of(k) * _BLOCK
            return pltpu.make_async_copy(
                bufs[b].at[pl.ds(0, _BLOCK)], o_hbm.at[pl.ds(base, _BLOCK)], out_sems[b]
            )

        for k in range(max_blocks + 1):
            if k < max_blocks:
                b = k % 2

                @pl.when(blk_of(k) < nb_full)
                def _start_in():
                    if k >= 2:
                        out_copy(k - 2, b).wait()
                    in_copy(k, b).start()

            if k >= 1:
                p = (k - 1) % 2

                @pl.when(blk_of(k - 1) < nb_full)
                def _process():
                    in_copy(k - 1, p).wait()
                    add_vectors(bufs[p], _BLOCK // _LANES)
                    out_copy(k - 1, p).start()

        for k in (max_blocks - 2, max_blocks - 1):
            if k >= 0:

                @pl.when(blk_of(k) < nb_full)
                def _drain():
                    out_copy(k, k % 2).wait()

        if tail:

            @pl.when(wid == _NUM_WORKERS - 1)
            def _process_tail():
                base = nb_full * _BLOCK
                pltpu.sync_copy(x_hbm.at[pl.ds(base, tail)], buf0.at[pl.ds(0, tail)])
                add_vectors(buf0, tail_vecs)
                pltpu.sync_copy(buf0.at[pl.ds(0, tail)], o_hbm.at[pl.ds(base, tail)])

    return run


def kernel(indices, values):
    nnz = indices.shape[0]
    if indices.dtype == jnp.int32:
        total = nnz * 2
        flat = indices.T.reshape(total)
        out = _build_sc_offset_add(total, False)(flat)
        return out.reshape(2, nnz).T, values
    words = lax.bitcast_convert_type(indices, jnp.int32)
    total = nnz * 4
    flat = words.reshape(total)
    out = _build_sc_offset_add(total, True)(flat)
    new_indices = lax.bitcast_convert_type(out.reshape(nnz, 2, 2), indices.dtype)
    return new_indices, values


# speedup vs baseline: 86.8927x; 15.3758x over previous
import functools

import jax
import jax.numpy as jnp
from jax import lax
from jax.experimental import pallas as pl
from jax.experimental.pallas import tpu as pltpu
from jax.experimental.pallas import tpu_sc as plsc

_START_IDX = 16384
_LANES = 16
_NUM_CORES = 2
_NUM_SUBCORES = 16
_NUM_WORKERS = _NUM_CORES * _NUM_SUBCORES
_TILE = 128
_TB = 128


@functools.lru_cache(maxsize=None)
def _build_sc_offset_add_tiles(num_tiles: int):
    nb_full = num_tiles // _TB
    tail = num_tiles - nb_full * _TB
    max_blocks = -(-nb_full // _NUM_WORKERS)
    mesh = plsc.VectorSubcoreMesh(core_axis_name="c", subcore_axis_name="s")

    @functools.partial(
        pl.kernel,
        out_type=jax.ShapeDtypeStruct((num_tiles, 2, _TILE), jnp.int32),
        mesh=mesh,
        scratch_types=[
            pltpu.VMEM((_TB, 2, _TILE), jnp.int32),
            pltpu.VMEM((_TB, 2, _TILE), jnp.int32),
            pltpu.SemaphoreType.DMA,
            pltpu.SemaphoreType.DMA,
            pltpu.SemaphoreType.DMA,
            pltpu.SemaphoreType.DMA,
        ],
    )
    def run(x_hbm, o_hbm, buf0, buf1, in0, in1, out0, out1):
        wid = lax.axis_index("s") * _NUM_CORES + lax.axis_index("c")
        bufs = [buf0, buf1]
        in_sems = [in0, in1]
        out_sems = [out0, out1]
        pat = jnp.full((_LANES,), _START_IDX, dtype=jnp.int32)

        def add_rows(buf, nrows):
            @plsc.parallel_loop(0, nrows, unroll=2)
            def _(t):
                for c in range(2):
                    for r in range(_TILE // _LANES):
                        sl = pl.ds(r * _LANES, _LANES)
                        buf[t, c, sl] = buf[t, c, sl] + pat

        def blk_of(k):
            return wid + k * _NUM_WORKERS

        def in_copy(k, b):
            return pltpu.make_async_copy(
                x_hbm.at[pl.ds(blk_of(k) * _TB, _TB)], bufs[b], in_sems[b]
            )

        def out_copy(k, b):
            return pltpu.make_async_copy(
                bufs[b], o_hbm.at[pl.ds(blk_of(k) * _TB, _TB)], out_sems[b]
            )

        for k in range(max_blocks + 1):
            if k < max_blocks:
                b = k % 2

                @pl.when(blk_of(k) < nb_full)
                def _start_in():
                    if k >= 2:
                        out_copy(k - 2, b).wait()
                    in_copy(k, b).start()

            if k >= 1:
                p = (k - 1) % 2

                @pl.when(blk_of(k - 1) < nb_full)
                def _process():
                    in_copy(k - 1, p).wait()
                    add_rows(bufs[p], _TB)
                    out_copy(k - 1, p).start()

        for k in (max_blocks - 2, max_blocks - 1):
            if k >= 0:

                @pl.when(blk_of(k) < nb_full)
                def _drain():
                    out_copy(k, k % 2).wait()

        if tail:

            @pl.when(wid == _NUM_WORKERS - 1)
            def _process_tail():
                base = nb_full * _TB
                pltpu.sync_copy(
                    x_hbm.at[pl.ds(base, tail)], buf0.at[pl.ds(0, tail)]
                )
                add_rows(buf0, tail)
                pltpu.sync_copy(
                    buf0.at[pl.ds(0, tail)], o_hbm.at[pl.ds(base, tail)]
                )

    return run


@functools.lru_cache(maxsize=None)
def _build_sc_offset_add_flat(total: int):
    block = 65520
    buf_words = 65528
    nb_full = total // block
    tail = total - nb_full * block
    max_blocks = -(-nb_full // _NUM_WORKERS)
    tail_vecs = -(-tail // _LANES)
    mesh = plsc.VectorSubcoreMesh(core_axis_name="c", subcore_axis_name="s")

    @functools.partial(
        pl.kernel,
        out_type=jax.ShapeDtypeStruct((total,), jnp.int32),
        mesh=mesh,
        scratch_types=[pltpu.VMEM((buf_words,), jnp.int32)],
    )
    def run(x_hbm, o_hbm, buf):
        wid = lax.axis_index("s") * _NUM_CORES + lax.axis_index("c")
        lane = lax.iota(jnp.int32, _LANES)
        pat = jnp.where(lane % 2 == 0, _START_IDX, 0)

        def add_vectors(nvec):
            @plsc.parallel_loop(0, nvec * _LANES, _LANES, unroll=8)
            def _(i):
                sl = pl.ds(i, _LANES)
                buf[sl] = buf[sl] + pat

        for k in range(max_blocks):
            blk = wid + k * _NUM_WORKERS

            @pl.when(blk < nb_full)
            def _process():
                base = blk * block
                pltpu.sync_copy(x_hbm.at[pl.ds(base, block)], buf.at[pl.ds(0, block)])
                add_vectors(block // _LANES)
                pltpu.sync_copy(buf.at[pl.ds(0, block)], o_hbm.at[pl.ds(base, block)])

        if tail:

            @pl.when(wid == _NUM_WORKERS - 1)
            def _process_tail():
                base = nb_full * block
                pltpu.sync_copy(x_hbm.at[pl.ds(base, tail)], buf.at[pl.ds(0, tail)])
                add_vectors(tail_vecs)
                pltpu.sync_copy(buf.at[pl.ds(0, tail)], o_hbm.at[pl.ds(base, tail)])

    return run


def kernel(indices, values):
    nnz = indices.shape[0]
    if indices.dtype == jnp.int32:
        rows_pad = -(-nnz // _TILE) * _TILE
        num_tiles = rows_pad // _TILE
        padded = jnp.pad(indices, ((0, rows_pad - nnz), (0, 0)))
        tiles = padded.reshape(num_tiles, _TILE, 2).transpose(0, 2, 1)
        out = _build_sc_offset_add_tiles(num_tiles)(tiles)
        back = out.transpose(0, 2, 1).reshape(rows_pad, 2)
        return back[:nnz], values
    words = lax.bitcast_convert_type(indices, jnp.int32)
    total = nnz * 4
    flat = words.reshape(total)
    out = _build_sc_offset_add_flat(total)(flat)
    new_indices = lax.bitcast_convert_type(out.reshape(nnz, 2, 2), indices.dtype)
    return new_indices, values
